# baseline (speedup 1.0000x reference)
import jax
import jax.numpy as jnp
from jax import lax
from jax.experimental import pallas as pl
from jax.experimental.pallas import tpu as pltpu
from jax.experimental.pallas import tpu_sc as plsc

N = 100000
HIDDEN = 128
VOCAB = 64
CHUNK = 128
NC = 2
NS = 16
NW = NC * NS
LANES = 16
CPW = 25
RPW = CPW * CHUNK
NPAD = NW * RPW
NBUF = 3


def _pair_body(w0, w1, w2, w3, t01, t23):
    t01[...] = w0[...][:, None, :] + w1[...][None, :, :]
    t23[...] = w2[...][:, None, :] + w3[...][None, :, :]


def _build_pair_tables(W0, W1, W2, W3):
    t01, t23 = pl.pallas_call(
        _pair_body,
        out_shape=[
            jax.ShapeDtypeStruct((VOCAB, VOCAB, HIDDEN), jnp.float32),
            jax.ShapeDtypeStruct((VOCAB, VOCAB, HIDDEN), jnp.float32),
        ],
    )(W0, W1, W2, W3)
    return (t01.reshape(VOCAB * VOCAB, HIDDEN),
            t23.reshape(VOCAB * VOCAB, HIDDEN))


def _sc_body(x0, x1, x2, x3, t01, t23, out,
             xa, xb, i01, i23,
             b01_0, b01_1, b01_2, b23_0, b23_1, b23_2,
             gs0, gs1, gs2, ws0, ws1, ws2):
    b01s = (b01_0, b01_1, b01_2)
    b23s = (b23_0, b23_1, b23_2)
    gsem = (gs0, gs1, gs2)
    wsem = (ws0, ws1, ws2)

    wid = lax.axis_index("s") * NC + lax.axis_index("c")
    base = pl.multiple_of(wid * RPW, RPW)

    pltpu.sync_copy(x0.at[pl.ds(base, RPW)], xa)
    pltpu.sync_copy(x1.at[pl.ds(base, RPW)], xb)

    def fold(j, c):
        s = pl.ds(pl.multiple_of(j * LANES, LANES), LANES)
        i01[s] = xa[s] * VOCAB + xb[s]
        return c

    lax.fori_loop(0, RPW // LANES, fold, 0)
    pltpu.sync_copy(x2.at[pl.ds(base, RPW)], xa)
    pltpu.sync_copy(x3.at[pl.ds(base, RPW)], xb)

    def fold2(j, c):
        s = pl.ds(pl.multiple_of(j * LANES, LANES), LANES)
        i23[s] = xa[s] * VOCAB + xb[s]
        return c

    lax.fori_loop(0, RPW // LANES, fold2, 0)

    def fire(c, p):
        s = pl.ds(c * CHUNK, CHUNK)
        pltpu.async_copy(t01.at[i01.at[s]], b01s[p], gsem[p])
        pltpu.async_copy(t23.at[i23.at[s]], b23s[p], gsem[p])

    def wait_g(p):
        pltpu.make_async_copy(t01.at[pl.ds(0, CHUNK)], b01s[p], gsem[p]).wait()
        pltpu.make_async_copy(t01.at[pl.ds(0, CHUNK)], b23s[p], gsem[p]).wait()

    def fire_w(c, p):
        pltpu.async_copy(b01s[p], out.at[pl.ds(base + c * CHUNK, CHUNK)],
                         wsem[p])

    def wait_w(p):
        pltpu.make_async_copy(b01s[p], out.at[pl.ds(0, CHUNK)], wsem[p]).wait()

    def add_block(p):
        b01, b23 = b01s[p], b23s[p]

        def add_row(r, c2):
            for j in range(HIDDEN // LANES):
                s = pl.ds(j * LANES, LANES)
                b01[r, s] = b01[r, s] + b23[r, s]
            return c2

        lax.fori_loop(0, CHUNK, add_row, 0)

    def step(c, p):
        pn = (p + 1) % NBUF

        @pl.when(c >= NBUF - 1)
        def _():
            wait_w(pn)

        fire(c + 1, pn)
        wait_g(p)
        add_block(p)
        fire_w(c, p)

    fire(0, 0)

    def ring(k3, carry):
        c0 = k3 * NBUF
        for j in range(NBUF):
            step(c0 + j, j)
        return carry

    lax.fori_loop(0, (CPW - 1) // NBUF, ring, 0)
    c_last = CPW - 1
    p_last = c_last % NBUF
    wait_g(p_last)
    add_block(p_last)
    fire_w(c_last, p_last)
    for p in range(NBUF):
        wait_w(p)


def kernel(x, W0, W1, W2, W3):
    xT = jnp.pad(x.astype(jnp.int32).T, ((0, 0), (0, NPAD - N)))
    x0, x1, x2, x3 = xT[0], xT[1], xT[2], xT[3]
    t01, t23 = _build_pair_tables(W0, W1, W2, W3)
    mesh = plsc.VectorSubcoreMesh(core_axis_name="c", subcore_axis_name="s")
    f = pl.kernel(
        _sc_body,
        mesh=mesh,
        out_type=jax.ShapeDtypeStruct((NPAD, HIDDEN), jnp.float32),
        scratch_types=[
            pltpu.VMEM((RPW,), jnp.int32),
            pltpu.VMEM((RPW,), jnp.int32),
            pltpu.VMEM((RPW,), jnp.int32),
            pltpu.VMEM((RPW,), jnp.int32),
            pltpu.VMEM((CHUNK, HIDDEN), jnp.float32),
            pltpu.VMEM((CHUNK, HIDDEN), jnp.float32),
            pltpu.VMEM((CHUNK, HIDDEN), jnp.float32),
            pltpu.VMEM((CHUNK, HIDDEN), jnp.float32),
            pltpu.VMEM((CHUNK, HIDDEN), jnp.float32),
            pltpu.VMEM((CHUNK, HIDDEN), jnp.float32),
            pltpu.SemaphoreType.DMA,
            pltpu.SemaphoreType.DMA,
            pltpu.SemaphoreType.DMA,
            pltpu.SemaphoreType.DMA,
            pltpu.SemaphoreType.DMA,
            pltpu.SemaphoreType.DMA,
        ],
    )
    outp = f(x0, x1, x2, x3, t01, t23)
    return outp[:N]


# speedup vs baseline: 2.4841x; 2.4841x over previous
import jax
import jax.numpy as jnp
from jax import lax
from jax.experimental import pallas as pl
from jax.experimental.pallas import tpu as pltpu
from jax.experimental.pallas import tpu_sc as plsc

N = 100000
HIDDEN = 128
VOCAB = 64
CHUNK = 64
NC = 2
NS = 16
NW = NC * NS
LANES = 16
CPW = 50
RPW = CPW * CHUNK
NPAD = NW * RPW
NBUF = 3


def _pair_body(w0, w1, w2, w3, t01, t23):
    t01[...] = w0[...][:, None, :] + w1[...][None, :, :]
    t23[...] = w2[...][:, None, :] + w3[...][None, :, :]


def _build_pair_tables(W0, W1, W2, W3):
    t01, t23 = pl.pallas_call(
        _pair_body,
        out_shape=[
            jax.ShapeDtypeStruct((VOCAB, VOCAB, HIDDEN), jnp.float32),
            jax.ShapeDtypeStruct((VOCAB, VOCAB, HIDDEN), jnp.float32),
        ],
    )(W0, W1, W2, W3)
    return (t01.reshape(VOCAB * VOCAB, HIDDEN),
            t23.reshape(VOCAB * VOCAB, HIDDEN))


def _sc_body(x0, x1, x2, x3, t01, t23, out,
             sh01, sh23, xa, xb, i01, i23,
             b01_0, b01_1, b01_2, b23_0, b23_1, b23_2,
             gs0, gs1, gs2, ws0, ws1, ws2):
    b01s = (b01_0, b01_1, b01_2)
    b23s = (b23_0, b23_1, b23_2)
    gsem = (gs0, gs1, gs2)
    wsem = (ws0, ws1, ws2)

    sid = lax.axis_index("s")
    wid = sid * NC + lax.axis_index("c")
    base = pl.multiple_of(wid * RPW, RPW)

    seg = VOCAB * VOCAB // NS
    sseg = pl.ds(pl.multiple_of(sid * seg, seg), seg)
    pltpu.sync_copy(t01.at[sseg], sh01.at[sseg])
    pltpu.sync_copy(t23.at[sseg], sh23.at[sseg])
    plsc.subcore_barrier()

    pltpu.sync_copy(x0.at[pl.ds(base, RPW)], xa)
    pltpu.sync_copy(x1.at[pl.ds(base, RPW)], xb)

    def fold(j, c):
        s = pl.ds(pl.multiple_of(j * LANES, LANES), LANES)
        i01[s] = xa[s] * VOCAB + xb[s]
        return c

    lax.fori_loop(0, RPW // LANES, fold, 0)
    pltpu.sync_copy(x2.at[pl.ds(base, RPW)], xa)
    pltpu.sync_copy(x3.at[pl.ds(base, RPW)], xb)

    def fold2(j, c):
        s = pl.ds(pl.multiple_of(j * LANES, LANES), LANES)
        i23[s] = xa[s] * VOCAB + xb[s]
        return c

    lax.fori_loop(0, RPW // LANES, fold2, 0)

    def fire(c, p):
        s = pl.ds(c * CHUNK, CHUNK)
        pltpu.async_copy(sh01.at[i01.at[s]], b01s[p], gsem[p])
        pltpu.async_copy(sh23.at[i23.at[s]], b23s[p], gsem[p])

    def wait_g(p):
        pltpu.make_async_copy(t01.at[pl.ds(0, CHUNK)], b01s[p], gsem[p]).wait()
        pltpu.make_async_copy(t01.at[pl.ds(0, CHUNK)], b23s[p], gsem[p]).wait()

    def fire_w(c, p):
        pltpu.async_copy(b01s[p], out.at[pl.ds(base + c * CHUNK, CHUNK)],
                         wsem[p])

    def wait_w(p):
        pltpu.make_async_copy(b01s[p], out.at[pl.ds(0, CHUNK)], wsem[p]).wait()

    def add_block(p):
        b01, b23 = b01s[p], b23s[p]

        def add_row(r, c2):
            for j in range(HIDDEN // LANES):
                s = pl.ds(j * LANES, LANES)
                b01[r, s] = b01[r, s] + b23[r, s]
            return c2

        lax.fori_loop(0, CHUNK, add_row, 0)

    def step(c, p):
        @pl.when(c < CPW)
        def _():
            pn = (p + 1) % NBUF

            @pl.when(c + 1 < CPW)
            def _():
                @pl.when(c >= NBUF - 1)
                def _():
                    wait_w(pn)

                fire(c + 1, pn)

            wait_g(p)
            add_block(p)
            fire_w(c, p)

    fire(0, 0)

    def ring(k3, carry):
        c0 = k3 * NBUF
        for j in range(NBUF):
            step(c0 + j, j)
        return carry

    lax.fori_loop(0, (CPW + NBUF - 1) // NBUF, ring, 0)
    for p in range(NBUF):
        wait_w(p)


def kernel(x, W0, W1, W2, W3):
    xT = jnp.pad(x.astype(jnp.int32).T, ((0, 0), (0, NPAD - N)))
    x0, x1, x2, x3 = xT[0], xT[1], xT[2], xT[3]
    t01, t23 = _build_pair_tables(W0, W1, W2, W3)
    mesh = plsc.VectorSubcoreMesh(core_axis_name="c", subcore_axis_name="s")
    f = pl.kernel(
        _sc_body,
        mesh=mesh,
        out_type=jax.ShapeDtypeStruct((NPAD, HIDDEN), jnp.float32),
        scratch_types=[
            pltpu.VMEM_SHARED((VOCAB * VOCAB, HIDDEN), jnp.float32),
            pltpu.VMEM_SHARED((VOCAB * VOCAB, HIDDEN), jnp.float32),
            pltpu.VMEM((RPW,), jnp.int32),
            pltpu.VMEM((RPW,), jnp.int32),
            pltpu.VMEM((RPW,), jnp.int32),
            pltpu.VMEM((RPW,), jnp.int32),
            pltpu.VMEM((CHUNK, HIDDEN), jnp.float32),
            pltpu.VMEM((CHUNK, HIDDEN), jnp.float32),
            pltpu.VMEM((CHUNK, HIDDEN), jnp.float32),
            pltpu.VMEM((CHUNK, HIDDEN), jnp.float32),
            pltpu.VMEM((CHUNK, HIDDEN), jnp.float32),
            pltpu.VMEM((CHUNK, HIDDEN), jnp.float32),
            pltpu.SemaphoreType.DMA,
            pltpu.SemaphoreType.DMA,
            pltpu.SemaphoreType.DMA,
            pltpu.SemaphoreType.DMA,
            pltpu.SemaphoreType.DMA,
            pltpu.SemaphoreType.DMA,
        ],
    )
    outp = f(x0, x1, x2, x3, t01, t23)
    return outp[:N]


# speedup vs baseline: 3.3280x; 1.3397x over previous
import jax
import jax.numpy as jnp
from jax import lax
from jax.experimental import pallas as pl
from jax.experimental.pallas import tpu as pltpu
from jax.experimental.pallas import tpu_sc as plsc

N = 100000
HIDDEN = 128
VOCAB = 64
CHUNK = 64
NC = 2
NS = 16
NW = NC * NS
LANES = 16
NBUF = 3

NFULL = N // CHUNK
TAIL = N - NFULL * CHUNK
CQ, CR = divmod(NFULL, NW)
CMAX = CQ + 1
RSTAGE = CMAX * CHUNK
XPAD = NFULL * CHUNK + CHUNK


def _pair_body(w0, w1, w2, w3, t01, t23):
    t01[...] = w0[...][:, None, :] + w1[...][None, :, :]
    t23[...] = w2[...][:, None, :] + w3[...][None, :, :]


def _build_pair_tables(W0, W1, W2, W3):
    t01, t23 = pl.pallas_call(
        _pair_body,
        out_shape=[
            jax.ShapeDtypeStruct((VOCAB, VOCAB, HIDDEN), jnp.float32),
            jax.ShapeDtypeStruct((VOCAB, VOCAB, HIDDEN), jnp.float32),
        ],
    )(W0, W1, W2, W3)
    return (t01.reshape(VOCAB * VOCAB, HIDDEN),
            t23.reshape(VOCAB * VOCAB, HIDDEN))


def _sc_body(x0, x1, x2, x3, t01, t23, out,
             sh01, sh23, xa, xb, i01, i23,
             b01_0, b01_1, b01_2, b23_0, b23_1, b23_2,
             gs0, gs1, gs2, ws0, ws1, ws2):
    b01s = (b01_0, b01_1, b01_2)
    b23s = (b23_0, b23_1, b23_2)
    gsem = (gs0, gs1, gs2)
    wsem = (ws0, ws1, ws2)

    sid = lax.axis_index("s")
    wid = sid * NC + lax.axis_index("c")
    nmine = jnp.where(wid < CR, CMAX, CQ)
    start = jnp.where(wid < CR, wid * CMAX, CR + wid * CQ)
    base = pl.multiple_of(start * CHUNK, CHUNK)

    seg = VOCAB * VOCAB // NS
    sseg = pl.ds(pl.multiple_of(sid * seg, seg), seg)
    pltpu.sync_copy(t01.at[sseg], sh01.at[sseg])
    pltpu.sync_copy(t23.at[sseg], sh23.at[sseg])
    plsc.subcore_barrier()

    pltpu.sync_copy(x0.at[pl.ds(base, RSTAGE)], xa)
    pltpu.sync_copy(x1.at[pl.ds(base, RSTAGE)], xb)

    def fold(j, c):
        s = pl.ds(pl.multiple_of(j * LANES, LANES), LANES)
        i01[s] = xa[s] * VOCAB + xb[s]
        return c

    lax.fori_loop(0, RSTAGE // LANES, fold, 0)
    pltpu.sync_copy(x2.at[pl.ds(base, RSTAGE)], xa)
    pltpu.sync_copy(x3.at[pl.ds(base, RSTAGE)], xb)

    def fold2(j, c):
        s = pl.ds(pl.multiple_of(j * LANES, LANES), LANES)
        i23[s] = xa[s] * VOCAB + xb[s]
        return c

    lax.fori_loop(0, RSTAGE // LANES, fold2, 0)

    def fire(c, p):
        s = pl.ds(c * CHUNK, CHUNK)
        pltpu.async_copy(sh01.at[i01.at[s]], b01s[p], gsem[p])
        pltpu.async_copy(sh23.at[i23.at[s]], b23s[p], gsem[p])

    def wait_g(p):
        pltpu.make_async_copy(t01.at[pl.ds(0, CHUNK)], b01s[p], gsem[p]).wait()
        pltpu.make_async_copy(t01.at[pl.ds(0, CHUNK)], b23s[p], gsem[p]).wait()

    def fire_w(c, p):
        pltpu.async_copy(b01s[p], out.at[pl.ds(base + c * CHUNK, CHUNK)],
                         wsem[p])

    def wait_w(p):
        pltpu.make_async_copy(b01s[p], out.at[pl.ds(0, CHUNK)], wsem[p]).wait()

    def add_block(p):
        b01, b23 = b01s[p], b23s[p]

        def add_row(r, c2):
            for j in range(HIDDEN // LANES):
                s = pl.ds(j * LANES, LANES)
                b01[r, s] = b01[r, s] + b23[r, s]
            return c2

        lax.fori_loop(0, CHUNK, add_row, 0)

    def step(c, p):
        @pl.when(c < nmine)
        def _():
            pn = (p + 1) % NBUF

            @pl.when(c + 1 < nmine)
            def _():
                @pl.when(c >= NBUF - 1)
                def _():
                    wait_w(pn)

                fire(c + 1, pn)

            wait_g(p)
            add_block(p)
            fire_w(c, p)

    fire(0, 0)

    def ring(k3, carry):
        c0 = k3 * NBUF
        for j in range(NBUF):
            step(c0 + j, j)
        return carry

    lax.fori_loop(0, (CMAX + NBUF - 1) // NBUF, ring, 0)
    for p in range(NBUF):
        wait_w(p)

    @pl.when(wid == NW - 1)
    def _():
        fire(CQ, 0)
        wait_g(0)
        add_block(0)
        pltpu.sync_copy(b01s[0].at[pl.ds(0, TAIL)],
                        out.at[pl.ds(NFULL * CHUNK, TAIL)])


def kernel(x, W0, W1, W2, W3):
    xT = jnp.pad(x.astype(jnp.int32).T, ((0, 0), (0, XPAD - N)))
    x0, x1, x2, x3 = xT[0], xT[1], xT[2], xT[3]
    t01, t23 = _build_pair_tables(W0, W1, W2, W3)
    mesh = plsc.VectorSubcoreMesh(core_axis_name="c", subcore_axis_name="s")
    f = pl.kernel(
        _sc_body,
        mesh=mesh,
        out_type=jax.ShapeDtypeStruct((N, HIDDEN), jnp.float32),
        scratch_types=[
            pltpu.VMEM_SHARED((VOCAB * VOCAB, HIDDEN), jnp.float32),
            pltpu.VMEM_SHARED((VOCAB * VOCAB, HIDDEN), jnp.float32),
            pltpu.VMEM((RSTAGE,), jnp.int32),
            pltpu.VMEM((RSTAGE,), jnp.int32),
            pltpu.VMEM((RSTAGE,), jnp.int32),
            pltpu.VMEM((RSTAGE,), jnp.int32),
            pltpu.VMEM((CHUNK, HIDDEN), jnp.float32),
            pltpu.VMEM((CHUNK, HIDDEN), jnp.float32),
            pltpu.VMEM((CHUNK, HIDDEN), jnp.float32),
            pltpu.VMEM((CHUNK, HIDDEN), jnp.float32),
            pltpu.VMEM((CHUNK, HIDDEN), jnp.float32),
            pltpu.VMEM((CHUNK, HIDDEN), jnp.float32),
            pltpu.SemaphoreType.DMA,
            pltpu.SemaphoreType.DMA,
            pltpu.SemaphoreType.DMA,
            pltpu.SemaphoreType.DMA,
            pltpu.SemaphoreType.DMA,
            pltpu.SemaphoreType.DMA,
        ],
    )
    return f(x0, x1, x2, x3, t01, t23)
